# baseline (speedup 1.0000x reference)
import jax
import jax.numpy as jnp
from jax.experimental import pallas as pl
from jax.experimental.pallas import tpu as pltpu

B, N, L, D = 8, 9, 512, 768
S = N - 1
ROWS_PER_BATCH_IN = N * L
ROWS_PER_BATCH_OUT = S * L
CHUNK = 4096
CPB = ROWS_PER_BATCH_OUT // CHUNK


def _select_kernel(valid_ref, reps_in, mask_in, reps_out, mask_out):
    b = pl.program_id(0)
    v = valid_ref[b]

    @pl.when(v != 0)
    def _copy():
        reps_out[...] = reps_in[...]

    @pl.when(v == 0)
    def _zero():
        reps_out[...] = jnp.zeros_like(reps_out)

    @pl.when(b == 0)
    def _mask():
        for bb in range(B):
            vb = valid_ref[bb]
            mask_out[bb] = jnp.where(vb != 0, mask_in[bb, 1:], 0.0)


def kernel(token_reps, token_mask, valid_sentences):
    valid_i32 = valid_sentences.astype(jnp.int32)
    reps2d = token_reps.reshape(B * N * L, D)
    mask4 = token_mask.reshape(B, N, 1, L)

    reps_out, mask_out = pl.pallas_call(
        _select_kernel,
        grid=(B, CPB),
        in_specs=[
            pl.BlockSpec(memory_space=pltpu.MemorySpace.SMEM),
            pl.BlockSpec(
                (pl.Element(CHUNK), pl.Element(D)),
                lambda b, c: (
                    pl.multiple_of(b * ROWS_PER_BATCH_IN + L + c * CHUNK, 512),
                    0,
                ),
            ),
            pl.BlockSpec(
                (pl.Element(B), pl.Element(N), pl.Element(1), pl.Element(L)),
                lambda b, c: (0, 0, 0, 0),
            ),
        ],
        out_specs=[
            pl.BlockSpec((CHUNK, D), lambda b, c: (b * CPB + c, 0)),
            pl.BlockSpec((B, S, 1, L), lambda b, c: (0, 0, 0, 0)),
        ],
        out_shape=[
            jax.ShapeDtypeStruct((B * S * L, D), jnp.float32),
            jax.ShapeDtypeStruct((B, S, 1, L), jnp.float32),
        ],
    )(valid_i32, reps2d, mask4)

    return reps_out.reshape(B, S, L, D), mask_out.reshape(B, S, L)


# speedup vs baseline: 1.0012x; 1.0012x over previous
import jax
import jax.numpy as jnp
from jax.experimental import pallas as pl
from jax.experimental.pallas import tpu as pltpu

B, N, L, D = 8, 9, 512, 768
S = N - 1
RPB_IN = N * L
RPB_OUT = S * L
NBUF = 4


def _bounce_kernel(valid_ref, reps_in, mask_in, reps_out, mask_out,
                   b0, b1, b2, b3, mbuf, mobuf,
                   si0, si1, si2, si3, so0, so1, so2, so3, smi, smo):
    bufs = (b0, b1, b2, b3)
    sins = (si0, si1, si2, si3)
    souts = (so0, so1, so2, so3)
    mk = pltpu.make_async_copy

    def in_cp(k):
        return mk(reps_in.at[pl.ds(k * RPB_IN + L, RPB_OUT)], bufs[k % NBUF],
                  sins[k % NBUF])

    def out_cp(k):
        return mk(bufs[k % NBUF], reps_out.at[pl.ds(k * RPB_OUT, RPB_OUT)],
                  souts[k % NBUF])

    mk(mask_in, mbuf, smi).start()
    for k in range(NBUF):
        in_cp(k).start()

    mk(mask_in, mbuf, smi).wait()
    for bb in range(B):
        mobuf[bb] = jnp.where(valid_ref[bb] != 0, mbuf[bb, 1:], 0.0)
    mk(mobuf, mask_out, smo).start()

    for k in range(B):
        in_cp(k).wait()

        @pl.when(valid_ref[k] == 0)
        def _zero(k=k):
            bufs[k % NBUF][...] = jnp.zeros_like(bufs[k % NBUF])

        out_cp(k).start()
        if k + NBUF < B:
            out_cp(k).wait()
            in_cp(k + NBUF).start()

    for k in range(B - NBUF, B):
        out_cp(k).wait()
    mk(mobuf, mask_out, smo).wait()


def kernel(token_reps, token_mask, valid_sentences):
    valid_i32 = valid_sentences.astype(jnp.int32)
    reps2d = token_reps.reshape(B * N * L, D)
    mask4 = token_mask.reshape(B, N, 1, L)

    reps_out, mask_out = pl.pallas_call(
        _bounce_kernel,
        in_specs=[
            pl.BlockSpec(memory_space=pltpu.MemorySpace.SMEM),
            pl.BlockSpec(memory_space=pltpu.MemorySpace.HBM),
            pl.BlockSpec(memory_space=pltpu.MemorySpace.HBM),
        ],
        out_specs=[
            pl.BlockSpec(memory_space=pltpu.MemorySpace.HBM),
            pl.BlockSpec(memory_space=pltpu.MemorySpace.HBM),
        ],
        out_shape=[
            jax.ShapeDtypeStruct((B * S * L, D), jnp.float32),
            jax.ShapeDtypeStruct((B, S, 1, L), jnp.float32),
        ],
        scratch_shapes=(
            [pltpu.MemorySpace.VMEM((RPB_OUT, D), jnp.float32)] * NBUF
            + [pltpu.MemorySpace.VMEM((B, N, 1, L), jnp.float32),
               pltpu.MemorySpace.VMEM((B, S, 1, L), jnp.float32)]
            + [pltpu.SemaphoreType.DMA] * 10
        ),
    )(valid_i32, reps2d, mask4)

    return reps_out.reshape(B, S, L, D), mask_out.reshape(B, S, L)


# speedup vs baseline: 1.0012x; 1.0001x over previous
import jax
import jax.numpy as jnp
from jax.experimental import pallas as pl
from jax.experimental.pallas import tpu as pltpu

B, N, L, D = 8, 9, 512, 768
S = N - 1
RPB_IN = N * L
RPB_OUT = S * L
CH = 2048
CPB = RPB_OUT // CH
NCHUNK = B * CPB
NBUF = 8


def _bounce_kernel(valid_ref, reps_in, mask_in, reps_out, mask_out,
                   *refs):
    bufs = refs[:NBUF]
    mbuf, mobuf = refs[NBUF], refs[NBUF + 1]
    sins = refs[NBUF + 2:2 * NBUF + 2]
    souts = refs[2 * NBUF + 2:3 * NBUF + 2]
    smi, smo = refs[3 * NBUF + 2], refs[3 * NBUF + 3]
    mk = pltpu.make_async_copy

    def in_cp(k):
        bat, off = divmod(k, CPB)
        return mk(reps_in.at[pl.ds(bat * RPB_IN + L + off * CH, CH)],
                  bufs[k % NBUF], sins[k % NBUF])

    def out_cp(k):
        return mk(bufs[k % NBUF], reps_out.at[pl.ds(k * CH, CH)],
                  souts[k % NBUF])

    mk(mask_in, mbuf, smi).start()
    for k in range(NBUF):
        in_cp(k).start()

    mk(mask_in, mbuf, smi).wait()
    for bb in range(B):
        mobuf[bb] = jnp.where(valid_ref[bb] != 0, mbuf[bb, 1:], 0.0)
    mk(mobuf, mask_out, smo).start()

    for k in range(NCHUNK):
        in_cp(k).wait()

        @pl.when(valid_ref[k // CPB] == 0)
        def _zero(k=k):
            bufs[k % NBUF][...] = jnp.zeros_like(bufs[k % NBUF])

        out_cp(k).start()
        if k + NBUF < NCHUNK:
            out_cp(k).wait()
            in_cp(k + NBUF).start()

    for k in range(NCHUNK - NBUF, NCHUNK):
        out_cp(k).wait()
    mk(mobuf, mask_out, smo).wait()


def kernel(token_reps, token_mask, valid_sentences):
    valid_i32 = valid_sentences.astype(jnp.int32)
    reps2d = token_reps.reshape(B * N * L, D)
    mask4 = token_mask.reshape(B, N, 1, L)

    reps_out, mask_out = pl.pallas_call(
        _bounce_kernel,
        in_specs=[
            pl.BlockSpec(memory_space=pltpu.MemorySpace.SMEM),
            pl.BlockSpec(memory_space=pltpu.MemorySpace.HBM),
            pl.BlockSpec(memory_space=pltpu.MemorySpace.HBM),
        ],
        out_specs=[
            pl.BlockSpec(memory_space=pltpu.MemorySpace.HBM),
            pl.BlockSpec(memory_space=pltpu.MemorySpace.HBM),
        ],
        out_shape=[
            jax.ShapeDtypeStruct((B * S * L, D), jnp.float32),
            jax.ShapeDtypeStruct((B, S, 1, L), jnp.float32),
        ],
        scratch_shapes=(
            [pltpu.MemorySpace.VMEM((CH, D), jnp.float32)] * NBUF
            + [pltpu.MemorySpace.VMEM((B, N, 1, L), jnp.float32),
               pltpu.MemorySpace.VMEM((B, S, 1, L), jnp.float32)]
            + [pltpu.SemaphoreType.DMA] * (2 * NBUF + 2)
        ),
    )(valid_i32, reps2d, mask4)

    return reps_out.reshape(B, S, L, D), mask_out.reshape(B, S, L)


# speedup vs baseline: 1.0016x; 1.0003x over previous
import jax
import jax.numpy as jnp
from jax.experimental import pallas as pl
from jax.experimental.pallas import tpu as pltpu

B, N, L, D = 8, 9, 512, 768
S = N - 1
ROWS_PER_BATCH_IN = N * L
ROWS_PER_BATCH_OUT = S * L
CHUNK = 4096
CPB = ROWS_PER_BATCH_OUT // CHUNK


def _select_kernel(valid_ref, reps_in, mask_in, reps_out, mask_out):
    b = pl.program_id(0)
    v = valid_ref[b]

    @pl.when(v != 0)
    def _copy():
        reps_out[...] = reps_in[...]
        mask_out[...] = mask_in[...]

    @pl.when(v == 0)
    def _zero():
        reps_out[...] = jnp.zeros_like(reps_out)
        mask_out[...] = jnp.zeros_like(mask_out)


def kernel(token_reps, token_mask, valid_sentences):
    valid_i32 = valid_sentences.astype(jnp.int32)
    reps2d = token_reps.reshape(B * N * L, D)
    mask4 = token_mask.reshape(B, N, 1, L)

    reps_out, mask_out = pl.pallas_call(
        _select_kernel,
        grid=(B, CPB),
        in_specs=[
            pl.BlockSpec(memory_space=pltpu.MemorySpace.SMEM),
            pl.BlockSpec(
                (pl.Element(CHUNK), pl.Element(D)),
                lambda b, c: (
                    pl.multiple_of(b * ROWS_PER_BATCH_IN + L + c * CHUNK, 512),
                    0,
                ),
            ),
            pl.BlockSpec(
                (pl.Element(1), pl.Element(S), pl.Element(1), pl.Element(L)),
                lambda b, c: (b, 1, 0, 0),
            ),
        ],
        out_specs=[
            pl.BlockSpec((CHUNK, D), lambda b, c: (b * CPB + c, 0)),
            pl.BlockSpec((1, S, 1, L), lambda b, c: (b, 0, 0, 0)),
        ],
        out_shape=[
            jax.ShapeDtypeStruct((B * S * L, D), jnp.float32),
            jax.ShapeDtypeStruct((B, S, 1, L), jnp.float32),
        ],
    )(valid_i32, reps2d, mask4)

    return reps_out.reshape(B, S, L, D), mask_out.reshape(B, S, L)
